# Initial kernel scaffold; baseline (speedup 1.0000x reference)
import jax
import jax.numpy as jnp
from jax import lax
from jax.experimental import pallas as pl
from jax.experimental.pallas import tpu as pltpu
from jax.experimental.pallas import tpu_sc as plsc

_N = 10000
_NP = 10240
_E = 320000
_NTILES = 32
_EPT = 10240
_C = 128
_NCHUNK = _EPT // _C
_D = 128
_ROWS_PER_TILE = _NP // 16
_WCHUNK = 160


_BLK = 1024


def _k1_body(x_ref, w_ref, b_ref, hn_ref, nr_ref):
    h = jnp.dot(x_ref[...], w_ref[...], preferred_element_type=jnp.float32)
    h = jnp.maximum(h + b_ref[...], 0.0)
    nr = jnp.sqrt(jnp.sum(h * h, axis=1, keepdims=True))
    hn_ref[...] = h / jnp.clip(nr, 1e-12, None)
    nr_ref[...] = nr


def _input_matmul(x, w, b):
    grid = _NP // _BLK
    return pl.pallas_call(
        _k1_body,
        grid=(grid,),
        in_specs=[
            pl.BlockSpec((_BLK, _D), lambda i: (i, 0)),
            pl.BlockSpec((_D, _D), lambda i: (0, 0)),
            pl.BlockSpec((1, _D), lambda i: (0, 0)),
        ],
        out_specs=[
            pl.BlockSpec((_BLK, _D), lambda i: (i, 0)),
            pl.BlockSpec((_BLK, 1), lambda i: (i, 0)),
        ],
        out_shape=[
            jax.ShapeDtypeStruct((_NP, _D), jnp.float32),
            jax.ShapeDtypeStruct((_NP, 1), jnp.float32),
        ],
    )(x, w, b)


def _k3_body(acc_ref, den_ref, hn_ref, nr_ref):
    a = acc_ref[...]
    d = den_ref[...]
    s = a[0] + a[1]
    dd = d[0, :, 0:1] + d[1, :, 0:1]
    h = jnp.maximum(s / jnp.where(dd > 0.0, dd, 1.0), 0.0)
    nr = jnp.sqrt(jnp.sum(h * h, axis=1, keepdims=True))
    hn_ref[...] = h / jnp.clip(nr, 1e-12, None)
    nr_ref[...] = nr


def _combine_normalize(acc, den):
    grid = _NP // _BLK
    return pl.pallas_call(
        _k3_body,
        grid=(grid,),
        in_specs=[
            pl.BlockSpec((2, _BLK, _D), lambda i: (0, i, 0)),
            pl.BlockSpec((2, _BLK, 8), lambda i: (0, i, 0)),
        ],
        out_specs=[
            pl.BlockSpec((_BLK, _D), lambda i: (i, 0)),
            pl.BlockSpec((_BLK, 1), lambda i: (i, 0)),
        ],
        out_shape=[
            jax.ShapeDtypeStruct((_NP, _D), jnp.float32),
            jax.ShapeDtypeStruct((_NP, 1), jnp.float32),
        ],
    )(acc, den)


def _k5_body(acc_ref, den_ref, w_ref, b_ref, out_ref):
    a = acc_ref[...]
    d = den_ref[...]
    s = a[0] + a[1]
    dd = d[0, :, 0:1] + d[1, :, 0:1]
    h = jnp.maximum(s / jnp.where(dd > 0.0, dd, 1.0), 0.0)
    z = jnp.dot(h, w_ref[...], preferred_element_type=jnp.float32) + b_ref[...]
    m = jnp.max(z, axis=1, keepdims=True)
    zz = z - m
    out_ref[...] = zz - jnp.log(jnp.sum(jnp.exp(zz), axis=1, keepdims=True))


def _output_head(acc, den, w2, b2, out_size):
    grid = _NP // _BLK
    return pl.pallas_call(
        _k5_body,
        grid=(grid,),
        in_specs=[
            pl.BlockSpec((2, _BLK, _D), lambda i: (0, i, 0)),
            pl.BlockSpec((2, _BLK, 8), lambda i: (0, i, 0)),
            pl.BlockSpec((_D, out_size), lambda i: (0, 0)),
            pl.BlockSpec((1, out_size), lambda i: (0, 0)),
        ],
        out_specs=pl.BlockSpec((_BLK, out_size), lambda i: (i, 0)),
        out_shape=jax.ShapeDtypeStruct((_NP, out_size), jnp.float32),
    )(acc, den, w2, b2)


def _sc_body(hn_hbm, nrm_hbm, src_hbm, dst_hbm, beta_hbm, z128_hbm, z8_hbm,
             acc_out, den_out,
             acc_sh, den_sh, src_v, dst_v, nrm_v, beta_v, srow, drow, exb,
             zb, zb8, sem_a, sem_b):
    c = lax.axis_index("c")
    s = lax.axis_index("s")
    wid = c * 16 + s

    pltpu.sync_copy(src_hbm.at[wid], src_v)
    pltpu.sync_copy(dst_hbm.at[wid], dst_v)
    pltpu.sync_copy(nrm_hbm, nrm_v)
    pltpu.sync_copy(beta_hbm, beta_v)

    pltpu.sync_copy(z128_hbm, zb)
    pltpu.sync_copy(z8_hbm, zb8)
    base = s * _ROWS_PER_TILE
    for k in range(_ROWS_PER_TILE // _WCHUNK):
        pltpu.sync_copy(zb, acc_sh.at[pl.ds(base + k * _WCHUNK, _WCHUNK)])
    pltpu.sync_copy(zb8, den_sh.at[pl.ds(base, _ROWS_PER_TILE)])
    pltpu.sync_copy(z8_hbm.at[pl.ds(0, _C)], exb)
    plsc.subcore_barrier()

    zero16 = jnp.zeros((16,), jnp.int32)

    @pl.loop(0, _NCHUNK)
    def _chunk(ch):
        cp_s = pltpu.async_copy(hn_hbm.at[src_v.at[ch]], srow, sem_a)
        cp_d = pltpu.async_copy(hn_hbm.at[dst_v.at[ch]], drow, sem_b)
        cp_s.wait()
        cp_d.wait()

        @pl.loop(0, _C // 16)
        def _grp(j):
            eids = j * 16 + lax.iota(jnp.int32, 16)
            acc = jnp.zeros((16,), jnp.float32)
            for f in range(_D):
                fv = jnp.full((16,), f, jnp.int32)
                acc += (plsc.load_gather(srow, [eids, fv]) *
                        plsc.load_gather(drow, [eids, fv]))
            ex = jnp.exp(beta_v[...] * acc)
            srcids = src_v[ch, pl.ds(j * 16, 16)]
            w = ex * plsc.load_gather(nrm_v, [srcids])
            plsc.store_scatter(exb, [eids, zero16], ex)
            for f in range(_D):
                fv = jnp.full((16,), f, jnp.int32)
                v = plsc.load_gather(srow, [eids, fv]) * w
                plsc.store_scatter(srow, [eids, fv], v)

        pltpu.sync_copy(srow, acc_sh.at[dst_v.at[ch]], add=True)
        pltpu.sync_copy(exb, den_sh.at[dst_v.at[ch]], add=True)

    plsc.subcore_barrier()

    for k in range(_ROWS_PER_TILE // _WCHUNK):
        pltpu.sync_copy(acc_sh.at[pl.ds(base + k * _WCHUNK, _WCHUNK)], zb)
        pltpu.sync_copy(zb, acc_out.at[c, pl.ds(base + k * _WCHUNK, _WCHUNK)])
    pltpu.sync_copy(den_sh.at[pl.ds(base, _ROWS_PER_TILE)], zb8)
    pltpu.sync_copy(zb8, den_out.at[c, pl.ds(base, _ROWS_PER_TILE)])


_sc_edge_pass = pl.kernel(
    _sc_body,
    out_type=(
        jax.ShapeDtypeStruct((2, _NP, _D), jnp.float32),
        jax.ShapeDtypeStruct((2, _NP, 8), jnp.float32),
    ),
    mesh=plsc.VectorSubcoreMesh(core_axis_name="c", subcore_axis_name="s"),
    scratch_types=[
        pltpu.VMEM_SHARED((_NP, _D), jnp.float32),
        pltpu.VMEM_SHARED((_NP, 8), jnp.float32),
        pltpu.VMEM((_NCHUNK, _C), jnp.int32),
        pltpu.VMEM((_NCHUNK, _C), jnp.int32),
        pltpu.VMEM((_NP,), jnp.float32),
        pltpu.VMEM((16,), jnp.float32),
        pltpu.VMEM((_C, _D), jnp.float32),
        pltpu.VMEM((_C, _D), jnp.float32),
        pltpu.VMEM((_C, 8), jnp.float32),
        pltpu.VMEM((_WCHUNK, _D), jnp.float32),
        pltpu.VMEM((_ROWS_PER_TILE, 8), jnp.float32),
        pltpu.SemaphoreType.DMA,
        pltpu.SemaphoreType.DMA,
    ],
)


@jax.jit
def kernel(features, edge_index, W1, b1, betas, W2, b2):
    out_size = W2.shape[1]
    x = jnp.pad(features, ((0, _NP - _N), (0, 0)))

    n_pad_edges = _NTILES * _EPT - _E
    pad_ids = jnp.full((n_pad_edges,), _NP - 1, dtype=jnp.int32)
    src = jnp.concatenate([edge_index[0], pad_ids]).reshape(_NTILES, _NCHUNK, _C)
    dst = jnp.concatenate([edge_index[1], pad_ids]).reshape(_NTILES, _NCHUNK, _C)

    z128 = jnp.zeros((_WCHUNK, _D), jnp.float32)
    z8 = jnp.zeros((_ROWS_PER_TILE, 8), jnp.float32)

    hn, nr = _input_matmul(x, W1, b1.reshape(1, -1))
    acc, den = _sc_edge_pass(hn, nr.reshape(-1), src, dst,
                             jnp.full((16,), betas[0], jnp.float32), z128, z8)
    hn2, nr2 = _combine_normalize(acc, den)
    acc2, den2 = _sc_edge_pass(hn2, nr2.reshape(-1), src, dst,
                               jnp.full((16,), betas[1], jnp.float32), z128, z8)
    out = _output_head(acc2, den2, W2, b2.reshape(1, -1), out_size)
    return out[:_N]


# speedup vs baseline: 5.9990x; 5.9990x over previous
import jax
import jax.numpy as jnp
from jax import lax
from jax.experimental import pallas as pl
from jax.experimental.pallas import tpu as pltpu
from jax.experimental.pallas import tpu_sc as plsc

_N = 10000
_NP = 10240
_E = 320000
_NTILES = 32
_EPT = 10240
_C = 64
_NCHUNK = _EPT // _C
_IB = 16
_NIB = _NCHUNK // _IB
_D = 128
_AW = 144
_ROWS_PER_TILE = _NP // 16
_WCHUNK = 40


_BLK = 1024


def _aug(h, aug_ref):
    nr = jnp.sqrt(jnp.sum(h * h, axis=1, keepdims=True))
    invn = 1.0 / jnp.clip(nr, 1e-12, None)
    aug_ref[...] = jnp.concatenate(
        [h, jnp.broadcast_to(invn, (h.shape[0], _AW - _D))], axis=1)


def _k1_body(x_ref, w_ref, b_ref, aug_ref):
    h = jnp.dot(x_ref[...], w_ref[...], preferred_element_type=jnp.float32)
    h = jnp.maximum(h + b_ref[...], 0.0)
    _aug(h, aug_ref)


def _input_matmul(x, w, b):
    return pl.pallas_call(
        _k1_body,
        grid=(_NP // _BLK,),
        in_specs=[
            pl.BlockSpec((_BLK, _D), lambda i: (i, 0)),
            pl.BlockSpec((_D, _D), lambda i: (0, 0)),
            pl.BlockSpec((1, _D), lambda i: (0, 0)),
        ],
        out_specs=pl.BlockSpec((_BLK, _AW), lambda i: (i, 0)),
        out_shape=jax.ShapeDtypeStruct((_NP, _AW), jnp.float32),
    )(x, w, b)


def _combine(acc_ref):
    a = acc_ref[...]
    s = a[0] + a[1]
    dd = s[:, _D:_D + 1]
    return jnp.maximum(s[:, :_D] / jnp.where(dd > 0.0, dd, 1.0), 0.0)


def _k3_body(acc_ref, aug_ref):
    _aug(_combine(acc_ref), aug_ref)


def _combine_normalize(acc):
    return pl.pallas_call(
        _k3_body,
        grid=(_NP // _BLK,),
        in_specs=[pl.BlockSpec((2, _BLK, _AW), lambda i: (0, i, 0))],
        out_specs=pl.BlockSpec((_BLK, _AW), lambda i: (i, 0)),
        out_shape=jax.ShapeDtypeStruct((_NP, _AW), jnp.float32),
    )(acc)


def _k5_body(acc_ref, w_ref, b_ref, out_ref):
    h = _combine(acc_ref)
    z = jnp.dot(h, w_ref[...], preferred_element_type=jnp.float32) + b_ref[...]
    m = jnp.max(z, axis=1, keepdims=True)
    zz = z - m
    out_ref[...] = zz - jnp.log(jnp.sum(jnp.exp(zz), axis=1, keepdims=True))


def _output_head(acc, w2, b2, out_size):
    return pl.pallas_call(
        _k5_body,
        grid=(_NP // _BLK,),
        in_specs=[
            pl.BlockSpec((2, _BLK, _AW), lambda i: (0, i, 0)),
            pl.BlockSpec((_D, out_size), lambda i: (0, 0)),
            pl.BlockSpec((1, out_size), lambda i: (0, 0)),
        ],
        out_specs=pl.BlockSpec((_BLK, out_size), lambda i: (i, 0)),
        out_shape=jax.ShapeDtypeStruct((_NP, out_size), jnp.float32),
    )(acc, w2, b2)


def _sc_body(aug_hbm, src_hbm, dst_hbm, beta_hbm, z_hbm,
             acc_out,
             acc_sh, sidx, didx, srow, drow, redbuf, wbuf, zb, beta_v,
             sem_a, sem_b):
    c = lax.axis_index("c")
    s = lax.axis_index("s")
    wid = c * 16 + s

    pltpu.sync_copy(beta_hbm, beta_v)

    pltpu.sync_copy(z_hbm, zb)
    base = s * _ROWS_PER_TILE
    for k in range(_ROWS_PER_TILE // _WCHUNK):
        pltpu.sync_copy(zb, acc_sh.at[pl.ds(base + k * _WCHUNK, _WCHUNK)])
    plsc.subcore_barrier()

    lanes16 = lax.iota(jnp.int32, 16) * 16

    @pl.loop(0, _NIB)
    def _iblock(ib):
        pltpu.sync_copy(src_hbm.at[wid, ib], sidx)
        pltpu.sync_copy(dst_hbm.at[wid, ib], didx)

        @pl.loop(0, _IB)
        def _chunk(ci):
            cp_s = pltpu.async_copy(aug_hbm.at[sidx.at[ci]], srow, sem_a)
            cp_d = pltpu.async_copy(aug_hbm.at[didx.at[ci]], drow, sem_b)
            cp_s.wait()
            cp_d.wait()

            @pl.loop(0, _C // 16)
            def _grp(j):
                e0 = j * 16
                for e in range(16):
                    acc = (srow[e0 + e, pl.ds(0, 16)] *
                           drow[e0 + e, pl.ds(0, 16)])
                    for k in range(1, _D // 16):
                        acc += (srow[e0 + e, pl.ds(k * 16, 16)] *
                                drow[e0 + e, pl.ds(k * 16, 16)])
                    acc = (acc * srow[e0 + e, pl.ds(_D, 16)] *
                           drow[e0 + e, pl.ds(_D, 16)])
                    redbuf[pl.ds(e * 16, 16)] = acc
                cos = plsc.load_gather(redbuf, [lanes16])
                for k in range(1, 16):
                    cos += plsc.load_gather(redbuf, [lanes16 + k])
                ex = jnp.exp(beta_v[...] * cos)
                wbuf[...] = ex
                for e in range(16):
                    exv = plsc.load_gather(
                        wbuf, [jnp.full((16,), e, jnp.int32)])
                    for k in range(_D // 16):
                        srow[e0 + e, pl.ds(k * 16, 16)] = (
                            srow[e0 + e, pl.ds(k * 16, 16)] * exv)
                    srow[e0 + e, pl.ds(_D, 16)] = exv

            pltpu.sync_copy(srow, acc_sh.at[didx.at[ci]], add=True)

    plsc.subcore_barrier()

    for k in range(_ROWS_PER_TILE // _WCHUNK):
        pltpu.sync_copy(acc_sh.at[pl.ds(base + k * _WCHUNK, _WCHUNK)], zb)
        pltpu.sync_copy(zb, acc_out.at[c, pl.ds(base + k * _WCHUNK, _WCHUNK)])


_sc_edge_pass = pl.kernel(
    _sc_body,
    out_type=jax.ShapeDtypeStruct((2, _NP, _AW), jnp.float32),
    mesh=plsc.VectorSubcoreMesh(core_axis_name="c", subcore_axis_name="s"),
    compiler_params=pltpu.CompilerParams(
        needs_layout_passes=False, use_tc_tiling_on_sc=False),
    scratch_types=[
        pltpu.VMEM_SHARED((_NP, _AW), jnp.float32),
        pltpu.VMEM((_IB, _C), jnp.int32),
        pltpu.VMEM((_IB, _C), jnp.int32),
        pltpu.VMEM((_C, _AW), jnp.float32),
        pltpu.VMEM((_C, _AW), jnp.float32),
        pltpu.VMEM((256,), jnp.float32),
        pltpu.VMEM((16,), jnp.float32),
        pltpu.VMEM((_WCHUNK, _AW), jnp.float32),
        pltpu.VMEM((16,), jnp.float32),
        pltpu.SemaphoreType.DMA,
        pltpu.SemaphoreType.DMA,
    ],
)


@jax.jit
def kernel(features, edge_index, W1, b1, betas, W2, b2):
    out_size = W2.shape[1]
    x = jnp.pad(features, ((0, _NP - _N), (0, 0)))

    n_pad_edges = _NTILES * _EPT - _E
    pad_ids = jnp.full((n_pad_edges,), _NP - 1, dtype=jnp.int32)
    src = jnp.concatenate([edge_index[0], pad_ids]).reshape(
        _NTILES, _NIB, _IB, _C)
    dst = jnp.concatenate([edge_index[1], pad_ids]).reshape(
        _NTILES, _NIB, _IB, _C)

    zrows = jnp.zeros((_WCHUNK, _AW), jnp.float32)

    aug1 = _input_matmul(x, W1, b1.reshape(1, -1))
    acc1 = _sc_edge_pass(aug1, src, dst,
                         jnp.full((16,), betas[0], jnp.float32), zrows)
    aug2 = _combine_normalize(acc1)
    acc2 = _sc_edge_pass(aug2, src, dst,
                         jnp.full((16,), betas[1], jnp.float32), zrows)
    return _output_head(acc2, W2, b2.reshape(1, -1), out_size)[:_N]


# speedup vs baseline: 7.3811x; 1.2304x over previous
import jax
import jax.numpy as jnp
from jax import lax
from jax.experimental import pallas as pl
from jax.experimental.pallas import tpu as pltpu
from jax.experimental.pallas import tpu_sc as plsc

_N = 10000
_NP = 10240
_E = 320000
_NTILES = 32
_EPT = 10240
_C = 32
_NCHUNK = _EPT // _C
_IB = 16
_NIB = _NCHUNK // _IB
_D = 128
_AW = 144
_ROWS_PER_TILE = _NP // 16
_WCHUNK = 40


_BLK = 1024


def _aug(h, aug_ref):
    nr = jnp.sqrt(jnp.sum(h * h, axis=1, keepdims=True))
    invn = 1.0 / jnp.clip(nr, 1e-12, None)
    aug_ref[...] = jnp.concatenate(
        [h, jnp.broadcast_to(invn, (h.shape[0], _AW - _D))], axis=1)


def _k1_body(x_ref, w_ref, b_ref, aug_ref):
    h = jnp.dot(x_ref[...], w_ref[...], preferred_element_type=jnp.float32)
    h = jnp.maximum(h + b_ref[...], 0.0)
    _aug(h, aug_ref)


def _input_matmul(x, w, b):
    return pl.pallas_call(
        _k1_body,
        grid=(_NP // _BLK,),
        in_specs=[
            pl.BlockSpec((_BLK, _D), lambda i: (i, 0)),
            pl.BlockSpec((_D, _D), lambda i: (0, 0)),
            pl.BlockSpec((1, _D), lambda i: (0, 0)),
        ],
        out_specs=pl.BlockSpec((_BLK, _AW), lambda i: (i, 0)),
        out_shape=jax.ShapeDtypeStruct((_NP, _AW), jnp.float32),
    )(x, w, b)


def _combine(acc_ref):
    a = acc_ref[...]
    s = a[0] + a[1]
    dd = s[:, _D:_D + 1]
    return jnp.maximum(s[:, :_D] / jnp.where(dd > 0.0, dd, 1.0), 0.0)


def _k3_body(acc_ref, aug_ref):
    _aug(_combine(acc_ref), aug_ref)


def _combine_normalize(acc):
    return pl.pallas_call(
        _k3_body,
        grid=(_NP // _BLK,),
        in_specs=[pl.BlockSpec((2, _BLK, _AW), lambda i: (0, i, 0))],
        out_specs=pl.BlockSpec((_BLK, _AW), lambda i: (i, 0)),
        out_shape=jax.ShapeDtypeStruct((_NP, _AW), jnp.float32),
    )(acc)


def _k5_body(acc_ref, w_ref, b_ref, out_ref):
    h = _combine(acc_ref)
    z = jnp.dot(h, w_ref[...], preferred_element_type=jnp.float32) + b_ref[...]
    m = jnp.max(z, axis=1, keepdims=True)
    zz = z - m
    out_ref[...] = zz - jnp.log(jnp.sum(jnp.exp(zz), axis=1, keepdims=True))


def _output_head(acc, w2, b2, out_size):
    return pl.pallas_call(
        _k5_body,
        grid=(_NP // _BLK,),
        in_specs=[
            pl.BlockSpec((2, _BLK, _AW), lambda i: (0, i, 0)),
            pl.BlockSpec((_D, out_size), lambda i: (0, 0)),
            pl.BlockSpec((1, out_size), lambda i: (0, 0)),
        ],
        out_specs=pl.BlockSpec((_BLK, out_size), lambda i: (i, 0)),
        out_shape=jax.ShapeDtypeStruct((_NP, out_size), jnp.float32),
    )(acc, w2, b2)


def _sc_body(aug_hbm, src_hbm, dst_hbm, beta_hbm, z_hbm,
             acc_out,
             acc_sh, sblk0, sblk1, dblk0, dblk1,
             srow0, srow1, srow2, srow3, drow0, drow1,
             redbuf, wbuf, zb, beta_v,
             gs, gd, gis, gid, sc0, sc1, sc2, sc3):
    c = lax.axis_index("c")
    s = lax.axis_index("s")
    wid = c * 16 + s
    sblk = (sblk0, sblk1)
    dblk = (dblk0, dblk1)
    srow = (srow0, srow1, srow2, srow3)
    drow = (drow0, drow1)
    sc = (sc0, sc1, sc2, sc3)

    pltpu.sync_copy(beta_hbm, beta_v)

    pltpu.sync_copy(z_hbm, zb)
    base = s * _ROWS_PER_TILE
    for k in range(_ROWS_PER_TILE // _WCHUNK):
        pltpu.sync_copy(zb, acc_sh.at[pl.ds(base + k * _WCHUNK, _WCHUNK)])
    plsc.subcore_barrier()

    lanes16 = lax.iota(jnp.int32, 16) * 16

    def compute(sr, dr):
        @pl.loop(0, _C // 16)
        def _grp(j):
            e0 = j * 16
            for e in range(16):
                acc = sr[e0 + e, pl.ds(0, 16)] * dr[e0 + e, pl.ds(0, 16)]
                for k in range(1, _D // 16):
                    acc += (sr[e0 + e, pl.ds(k * 16, 16)] *
                            dr[e0 + e, pl.ds(k * 16, 16)])
                acc = (acc * sr[e0 + e, pl.ds(_D, 16)] *
                       dr[e0 + e, pl.ds(_D, 16)])
                redbuf[pl.ds(e * 16, 16)] = acc
            cos = plsc.load_gather(redbuf, [lanes16])
            for k in range(1, 16):
                cos += plsc.load_gather(redbuf, [lanes16 + k])
            ex = jnp.exp(beta_v[...] * cos)
            wbuf[...] = ex
            for e in range(16):
                exv = plsc.load_gather(wbuf, [jnp.full((16,), e, jnp.int32)])
                for k in range(_D // 16):
                    sr[e0 + e, pl.ds(k * 16, 16)] = (
                        sr[e0 + e, pl.ds(k * 16, 16)] * exv)
                sr[e0 + e, pl.ds(_D, 16)] = exv

    def fire_gather(idx_s_ref, idx_d_ref, slot):
        pltpu.async_copy(aug_hbm.at[idx_s_ref], srow[slot], gs)
        pltpu.async_copy(aug_hbm.at[idx_d_ref], drow[slot % 2], gd)

    def wait_gather(slot):
        pltpu.make_async_copy(
            aug_hbm.at[sblk0.at[0]], srow[slot], gs).wait()
        pltpu.make_async_copy(
            aug_hbm.at[dblk0.at[0]], drow[slot % 2], gd).wait()

    def wait_scatter(slot):
        pltpu.make_async_copy(
            srow[slot], acc_sh.at[dblk0.at[0]], sc[slot]).wait()

    pltpu.sync_copy(src_hbm.at[wid, 0], sblk0)
    pltpu.sync_copy(dst_hbm.at[wid, 0], dblk0)
    fire_gather(sblk0.at[0], dblk0.at[0], 0)

    @pl.loop(0, _NIB // 2)
    def _outer(outer):
        for pb in range(2):
            cur_s, cur_d = sblk[pb], dblk[pb]
            nxt_s, nxt_d = sblk[1 - pb], dblk[1 - pb]

            @pl.loop(0, 4)
            def _quad(inner):
                for b in range(4):
                    sb = inner * 4 + b
                    wait_gather(b)
                    if b < 3:
                        if pb == 0:
                            @pl.when(jnp.logical_not(
                                (outer == 0) & (inner == 0)))
                            def _():
                                wait_scatter((b + 1) % 4)
                        else:
                            wait_scatter((b + 1) % 4)

                        if b == 0:
                            fetch_ok = (inner == 1) if pb == 0 else \
                                (inner == 1) & (outer < _NIB // 2 - 1)

                            @pl.when(fetch_ok)
                            def _():
                                nib = outer * 2 + pb + 1
                                pltpu.async_copy(
                                    src_hbm.at[wid, nib], nxt_s, gis)
                                pltpu.async_copy(
                                    dst_hbm.at[wid, nib], nxt_d, gid)

                        fire_gather(cur_s.at[sb + 1], cur_d.at[sb + 1],
                                    (b + 1) % 4)
                    else:
                        wait_scatter(0)

                        @pl.when(inner < 3)
                        def _():
                            fire_gather(cur_s.at[sb + 1], cur_d.at[sb + 1], 0)

                        blk_x = (inner == 3) if pb == 0 else \
                            (inner == 3) & (outer < _NIB // 2 - 1)

                        @pl.when(blk_x)
                        def _():
                            pltpu.make_async_copy(
                                src_hbm.at[wid, 0], nxt_s, gis).wait()
                            pltpu.make_async_copy(
                                dst_hbm.at[wid, 0], nxt_d, gid).wait()
                            fire_gather(nxt_s.at[0], nxt_d.at[0], 0)

                    compute(srow[b], drow[b % 2])
                    pltpu.async_copy(
                        srow[b], acc_sh.at[cur_d.at[sb]], sc[b], add=True)

    for slot in (1, 2, 3):
        wait_scatter(slot)

    plsc.subcore_barrier()

    for k in range(_ROWS_PER_TILE // _WCHUNK):
        pltpu.sync_copy(acc_sh.at[pl.ds(base + k * _WCHUNK, _WCHUNK)], zb)
        pltpu.sync_copy(zb, acc_out.at[c, pl.ds(base + k * _WCHUNK, _WCHUNK)])


_sc_edge_pass = pl.kernel(
    _sc_body,
    out_type=jax.ShapeDtypeStruct((2, _NP, _AW), jnp.float32),
    mesh=plsc.VectorSubcoreMesh(core_axis_name="c", subcore_axis_name="s"),
    compiler_params=pltpu.CompilerParams(
        needs_layout_passes=False, use_tc_tiling_on_sc=False),
    scratch_types=[
        pltpu.VMEM_SHARED((_NP, _AW), jnp.float32),
        pltpu.VMEM((_IB, _C), jnp.int32),
        pltpu.VMEM((_IB, _C), jnp.int32),
        pltpu.VMEM((_IB, _C), jnp.int32),
        pltpu.VMEM((_IB, _C), jnp.int32),
        pltpu.VMEM((_C, _AW), jnp.float32),
        pltpu.VMEM((_C, _AW), jnp.float32),
        pltpu.VMEM((_C, _AW), jnp.float32),
        pltpu.VMEM((_C, _AW), jnp.float32),
        pltpu.VMEM((_C, _AW), jnp.float32),
        pltpu.VMEM((_C, _AW), jnp.float32),
        pltpu.VMEM((256,), jnp.float32),
        pltpu.VMEM((16,), jnp.float32),
        pltpu.VMEM((_WCHUNK, _AW), jnp.float32),
        pltpu.VMEM((16,), jnp.float32),
        pltpu.SemaphoreType.DMA,
        pltpu.SemaphoreType.DMA,
        pltpu.SemaphoreType.DMA,
        pltpu.SemaphoreType.DMA,
        pltpu.SemaphoreType.DMA,
        pltpu.SemaphoreType.DMA,
        pltpu.SemaphoreType.DMA,
        pltpu.SemaphoreType.DMA,
    ],
)


@jax.jit
def kernel(features, edge_index, W1, b1, betas, W2, b2):
    out_size = W2.shape[1]
    x = jnp.pad(features, ((0, _NP - _N), (0, 0)))

    n_pad_edges = _NTILES * _EPT - _E
    pad_ids = jnp.full((n_pad_edges,), _NP - 1, dtype=jnp.int32)
    src = jnp.concatenate([edge_index[0], pad_ids]).reshape(
        _NTILES, _NIB, _IB, _C)
    dst = jnp.concatenate([edge_index[1], pad_ids]).reshape(
        _NTILES, _NIB, _IB, _C)

    zrows = jnp.zeros((_WCHUNK, _AW), jnp.float32)

    aug1 = _input_matmul(x, W1, b1.reshape(1, -1))
    acc1 = _sc_edge_pass(aug1, src, dst,
                         jnp.full((16,), betas[0], jnp.float32), zrows)
    aug2 = _combine_normalize(acc1)
    acc2 = _sc_edge_pass(aug2, src, dst,
                         jnp.full((16,), betas[1], jnp.float32), zrows)
    return _output_head(acc2, W2, b2.reshape(1, -1), out_size)[:_N]


# speedup vs baseline: 8.0209x; 1.0867x over previous
import jax
import jax.numpy as jnp
from jax import lax
from jax.experimental import pallas as pl
from jax.experimental.pallas import tpu as pltpu
from jax.experimental.pallas import tpu_sc as plsc

_N = 10000
_NP = 10240
_E = 320000
_NTILES = 32
_EPT = 10240
_C = 32
_NCHUNK = _EPT // _C
_IB = 8
_NIB = _NCHUNK // _IB
_D = 128
_AW = 144
_ROWS_PER_TILE = _NP // 16
_WCHUNK = 32


_BLK = 1024


def _aug(h, aug_ref):
    nr = jnp.sqrt(jnp.sum(h * h, axis=1, keepdims=True))
    invn = 1.0 / jnp.clip(nr, 1e-12, None)
    aug_ref[...] = jnp.concatenate(
        [h, jnp.broadcast_to(invn, (h.shape[0], _AW - _D))], axis=1)


def _k1_body(x_ref, w_ref, b_ref, aug_ref):
    h = jnp.dot(x_ref[...], w_ref[...], preferred_element_type=jnp.float32)
    h = jnp.maximum(h + b_ref[...], 0.0)
    _aug(h, aug_ref)


def _input_matmul(x, w, b):
    return pl.pallas_call(
        _k1_body,
        grid=(_NP // _BLK,),
        in_specs=[
            pl.BlockSpec((_BLK, _D), lambda i: (i, 0)),
            pl.BlockSpec((_D, _D), lambda i: (0, 0)),
            pl.BlockSpec((1, _D), lambda i: (0, 0)),
        ],
        out_specs=pl.BlockSpec((_BLK, _AW), lambda i: (i, 0)),
        out_shape=jax.ShapeDtypeStruct((_NP, _AW), jnp.float32),
    )(x, w, b)


def _combine(acc_ref):
    a = acc_ref[...]
    s = a[0] + a[1]
    dd = s[:, _D:_D + 1]
    return jnp.maximum(s[:, :_D] / jnp.where(dd > 0.0, dd, 1.0), 0.0)


def _k3_body(acc_ref, aug_ref):
    _aug(_combine(acc_ref), aug_ref)


def _combine_normalize(acc):
    return pl.pallas_call(
        _k3_body,
        grid=(_NP // _BLK,),
        in_specs=[pl.BlockSpec((2, _BLK, _AW), lambda i: (0, i, 0))],
        out_specs=pl.BlockSpec((_BLK, _AW), lambda i: (i, 0)),
        out_shape=jax.ShapeDtypeStruct((_NP, _AW), jnp.float32),
    )(acc)


def _k5_body(acc_ref, w_ref, b_ref, out_ref):
    h = _combine(acc_ref)
    z = jnp.dot(h, w_ref[...], preferred_element_type=jnp.float32) + b_ref[...]
    m = jnp.max(z, axis=1, keepdims=True)
    zz = z - m
    out_ref[...] = zz - jnp.log(jnp.sum(jnp.exp(zz), axis=1, keepdims=True))


def _output_head(acc, w2, b2, out_size):
    return pl.pallas_call(
        _k5_body,
        grid=(_NP // _BLK,),
        in_specs=[
            pl.BlockSpec((2, _BLK, _AW), lambda i: (0, i, 0)),
            pl.BlockSpec((_D, out_size), lambda i: (0, 0)),
            pl.BlockSpec((1, out_size), lambda i: (0, 0)),
        ],
        out_specs=pl.BlockSpec((_BLK, out_size), lambda i: (i, 0)),
        out_shape=jax.ShapeDtypeStruct((_NP, out_size), jnp.float32),
    )(acc, w2, b2)


def _sc_body(aug_hbm, src_hbm, dst_hbm, beta_hbm, z_hbm,
             acc_out,
             acc_sh, sblk0, sblk1, dblk0, dblk1,
             srow0, srow1, srow2, srow3, drow0, drow1, drow2, drow3,
             redbuf, wbuf, beta_v,
             gs, gd, gis, gid, sc0, sc1, sc2, sc3):
    c = lax.axis_index("c")
    s = lax.axis_index("s")
    wid = c * 16 + s
    sblk = (sblk0, sblk1)
    dblk = (dblk0, dblk1)
    srow = (srow0, srow1, srow2, srow3)
    drow = (drow0, drow1, drow2, drow3)
    sc = (sc0, sc1, sc2, sc3)

    pltpu.sync_copy(beta_hbm, beta_v)

    pltpu.sync_copy(z_hbm, srow0)
    base = s * _ROWS_PER_TILE
    for k in range(_ROWS_PER_TILE // _WCHUNK):
        pltpu.sync_copy(srow0, acc_sh.at[pl.ds(base + k * _WCHUNK, _WCHUNK)])
    plsc.subcore_barrier()

    lanes16 = lax.iota(jnp.int32, 16) * 16

    def compute(sr, dr):
        @pl.loop(0, _C // 16)
        def _grp(j):
            e0 = j * 16
            for e in range(16):
                acc = sr[e0 + e, pl.ds(0, 16)] * dr[e0 + e, pl.ds(0, 16)]
                for k in range(1, _D // 16):
                    acc += (sr[e0 + e, pl.ds(k * 16, 16)] *
                            dr[e0 + e, pl.ds(k * 16, 16)])
                acc = (acc * sr[e0 + e, pl.ds(_D, 16)] *
                       dr[e0 + e, pl.ds(_D, 16)])
                redbuf[pl.ds(e * 16, 16)] = acc
            cos = plsc.load_gather(redbuf, [lanes16])
            for k in range(1, 16):
                cos += plsc.load_gather(redbuf, [lanes16 + k])
            ex = jnp.exp(beta_v[...] * cos)
            wbuf[...] = ex
            for e in range(16):
                exv = plsc.load_gather(wbuf, [jnp.full((16,), e, jnp.int32)])
                for k in range(_D // 16):
                    sr[e0 + e, pl.ds(k * 16, 16)] = (
                        sr[e0 + e, pl.ds(k * 16, 16)] * exv)
                sr[e0 + e, pl.ds(_D, 16)] = exv

    def fire_gather(idx_s_ref, idx_d_ref, slot):
        pltpu.async_copy(aug_hbm.at[idx_s_ref], srow[slot], gs)
        pltpu.async_copy(aug_hbm.at[idx_d_ref], drow[slot], gd)

    def wait_gather(slot):
        pltpu.make_async_copy(
            aug_hbm.at[sblk0.at[0]], srow[slot], gs).wait()
        pltpu.make_async_copy(
            aug_hbm.at[dblk0.at[0]], drow[slot], gd).wait()

    def wait_scatter(slot):
        pltpu.make_async_copy(
            srow[slot], acc_sh.at[dblk0.at[0]], sc[slot]).wait()

    pltpu.sync_copy(src_hbm.at[wid, 0], sblk0)
    pltpu.sync_copy(dst_hbm.at[wid, 0], dblk0)
    fire_gather(sblk0.at[0], dblk0.at[0], 0)
    fire_gather(sblk0.at[1], dblk0.at[1], 1)

    @pl.loop(0, _NIB // 2)
    def _outer(outer):
        for pb in range(2):
            cur_s, cur_d = sblk[pb], dblk[pb]
            nxt_s, nxt_d = sblk[1 - pb], dblk[1 - pb]

            @pl.loop(0, _IB // 4)
            def _quad(inner):
                for b in range(4):
                    sb = inner * 4 + b
                    wait_gather(b)
                    if b == 0:
                        fetch_ok = (inner == 1) if pb == 0 else \
                            (inner == 1) & (outer < _NIB // 2 - 1)

                        @pl.when(fetch_ok)
                        def _():
                            nib = outer * 2 + pb + 1
                            pltpu.async_copy(
                                src_hbm.at[wid, nib], nxt_s, gis)
                            pltpu.async_copy(
                                dst_hbm.at[wid, nib], nxt_d, gid)

                    if b < 2:
                        if pb == 0:
                            @pl.when(jnp.logical_not(
                                (outer == 0) & (inner == 0)))
                            def _():
                                wait_scatter((b + 2) % 4)
                                fire_gather(cur_s.at[sb + 2],
                                            cur_d.at[sb + 2], (b + 2) % 4)

                            @pl.when((outer == 0) & (inner == 0))
                            def _():
                                fire_gather(cur_s.at[sb + 2],
                                            cur_d.at[sb + 2], (b + 2) % 4)
                        else:
                            wait_scatter((b + 2) % 4)
                            fire_gather(cur_s.at[sb + 2],
                                        cur_d.at[sb + 2], (b + 2) % 4)
                    else:
                        @pl.when(inner < _IB // 4 - 1)
                        def _():
                            wait_scatter((b + 2) % 4)
                            fire_gather(cur_s.at[sb + 2],
                                        cur_d.at[sb + 2], (b + 2) % 4)

                        blk_x = (inner == _IB // 4 - 1) if pb == 0 else \
                            (inner == _IB // 4 - 1) & (outer < _NIB // 2 - 1)

                        @pl.when(blk_x)
                        def _():
                            wait_scatter((b + 2) % 4)
                            if b == 2:
                                pltpu.make_async_copy(
                                    src_hbm.at[wid, 0], nxt_s, gis).wait()
                                pltpu.make_async_copy(
                                    dst_hbm.at[wid, 0], nxt_d, gid).wait()
                            fire_gather(nxt_s.at[b - 2], nxt_d.at[b - 2],
                                        (b + 2) % 4)

                    compute(srow[b], drow[b])
                    pltpu.async_copy(
                        srow[b], acc_sh.at[cur_d.at[sb]], sc[b], add=True)

    for slot in (0, 1, 2, 3):
        wait_scatter(slot)

    plsc.subcore_barrier()

    for k in range(_ROWS_PER_TILE // _WCHUNK):
        pltpu.sync_copy(acc_sh.at[pl.ds(base + k * _WCHUNK, _WCHUNK)], srow0)
        pltpu.sync_copy(srow0,
                        acc_out.at[c, pl.ds(base + k * _WCHUNK, _WCHUNK)])


_sc_edge_pass = pl.kernel(
    _sc_body,
    out_type=jax.ShapeDtypeStruct((2, _NP, _AW), jnp.float32),
    mesh=plsc.VectorSubcoreMesh(core_axis_name="c", subcore_axis_name="s"),
    compiler_params=pltpu.CompilerParams(
        needs_layout_passes=False, use_tc_tiling_on_sc=False),
    scratch_types=[
        pltpu.VMEM_SHARED((_NP, _AW), jnp.float32),
        pltpu.VMEM((_IB, _C), jnp.int32),
        pltpu.VMEM((_IB, _C), jnp.int32),
        pltpu.VMEM((_IB, _C), jnp.int32),
        pltpu.VMEM((_IB, _C), jnp.int32),
        pltpu.VMEM((_C, _AW), jnp.float32),
        pltpu.VMEM((_C, _AW), jnp.float32),
        pltpu.VMEM((_C, _AW), jnp.float32),
        pltpu.VMEM((_C, _AW), jnp.float32),
        pltpu.VMEM((_C, _AW), jnp.float32),
        pltpu.VMEM((_C, _AW), jnp.float32),
        pltpu.VMEM((_C, _AW), jnp.float32),
        pltpu.VMEM((_C, _AW), jnp.float32),
        pltpu.VMEM((256,), jnp.float32),
        pltpu.VMEM((16,), jnp.float32),
        pltpu.VMEM((16,), jnp.float32),
        pltpu.SemaphoreType.DMA,
        pltpu.SemaphoreType.DMA,
        pltpu.SemaphoreType.DMA,
        pltpu.SemaphoreType.DMA,
        pltpu.SemaphoreType.DMA,
        pltpu.SemaphoreType.DMA,
        pltpu.SemaphoreType.DMA,
        pltpu.SemaphoreType.DMA,
    ],
)


@jax.jit
def kernel(features, edge_index, W1, b1, betas, W2, b2):
    out_size = W2.shape[1]
    x = jnp.pad(features, ((0, _NP - _N), (0, 0)))

    n_pad_edges = _NTILES * _EPT - _E
    pad_ids = jnp.full((n_pad_edges,), _NP - 1, dtype=jnp.int32)
    src = jnp.concatenate([edge_index[0], pad_ids]).reshape(
        _NTILES, _NIB, _IB, _C)
    dst = jnp.concatenate([edge_index[1], pad_ids]).reshape(
        _NTILES, _NIB, _IB, _C)

    zrows = jnp.zeros((_C, _AW), jnp.float32)

    aug1 = _input_matmul(x, W1, b1.reshape(1, -1))
    acc1 = _sc_edge_pass(aug1, src, dst,
                         jnp.full((16,), betas[0], jnp.float32), zrows)
    aug2 = _combine_normalize(acc1)
    acc2 = _sc_edge_pass(aug2, src, dst,
                         jnp.full((16,), betas[1], jnp.float32), zrows)
    return _output_head(acc2, W2, b2.reshape(1, -1), out_size)[:_N]


# speedup vs baseline: 8.9064x; 1.1104x over previous
import jax
import jax.numpy as jnp
from jax import lax
from jax.experimental import pallas as pl
from jax.experimental.pallas import tpu as pltpu
from jax.experimental.pallas import tpu_sc as plsc

_N = 10000
_NP = 10240
_E = 320000
_NTILES = 32
_EPT = 10240
_C = 32
_NCHUNK = _EPT // _C
_IB = 8
_NIB = _NCHUNK // _IB
_D = 128
_AW = 144
_PW = 80
_ROWS_PER_TILE = _NP // 16
_WCHUNK = 32


_BLK = 1024


def _aug(h, aug_ref, augp_ref):
    nr = jnp.sqrt(jnp.sum(h * h, axis=1, keepdims=True))
    invn = 1.0 / jnp.clip(nr, 1e-12, None)
    invb = jnp.broadcast_to(invn, (h.shape[0], 16))
    aug_ref[...] = jnp.concatenate([h, invb], axis=1)
    lo = jnp.concatenate(
        [h[:, 32 * g:32 * g + 16] for g in range(4)], axis=1)
    hi = jnp.concatenate(
        [h[:, 32 * g + 16:32 * g + 32] for g in range(4)], axis=1)
    lo32 = lax.convert_element_type(
        lax.bitcast_convert_type(
            lax.convert_element_type(lo, jnp.bfloat16), jnp.uint16),
        jnp.uint32)
    hi32 = lax.convert_element_type(
        lax.bitcast_convert_type(
            lax.convert_element_type(hi, jnp.bfloat16), jnp.uint16),
        jnp.uint32)
    packed = lax.bitcast_convert_type((hi32 << 16) | lo32, jnp.float32)
    augp_ref[...] = jnp.concatenate([packed, invb], axis=1)


def _k1_body(x_ref, w_ref, b_ref, aug_ref, augp_ref):
    h = jnp.dot(x_ref[...], w_ref[...], preferred_element_type=jnp.float32)
    h = jnp.maximum(h + b_ref[...], 0.0)
    _aug(h, aug_ref, augp_ref)


_AUG_OUT = [
    jax.ShapeDtypeStruct((_NP, _AW), jnp.float32),
    jax.ShapeDtypeStruct((_NP, _PW), jnp.float32),
]
_AUG_SPECS = [
    pl.BlockSpec((_BLK, _AW), lambda i: (i, 0)),
    pl.BlockSpec((_BLK, _PW), lambda i: (i, 0)),
]


def _input_matmul(x, w, b):
    return pl.pallas_call(
        _k1_body,
        grid=(_NP // _BLK,),
        in_specs=[
            pl.BlockSpec((_BLK, _D), lambda i: (i, 0)),
            pl.BlockSpec((_D, _D), lambda i: (0, 0)),
            pl.BlockSpec((1, _D), lambda i: (0, 0)),
        ],
        out_specs=_AUG_SPECS,
        out_shape=_AUG_OUT,
    )(x, w, b)


def _combine(acc_ref):
    a = acc_ref[...]
    s = a[0] + a[1]
    dd = s[:, _D:_D + 1]
    return jnp.maximum(s[:, :_D] / jnp.where(dd > 0.0, dd, 1.0), 0.0)


def _k3_body(acc_ref, aug_ref, augp_ref):
    _aug(_combine(acc_ref), aug_ref, augp_ref)


def _combine_normalize(acc):
    return pl.pallas_call(
        _k3_body,
        grid=(_NP // _BLK,),
        in_specs=[pl.BlockSpec((2, _BLK, _AW), lambda i: (0, i, 0))],
        out_specs=_AUG_SPECS,
        out_shape=_AUG_OUT,
    )(acc)


def _k5_body(acc_ref, w_ref, b_ref, out_ref):
    h = _combine(acc_ref)
    z = jnp.dot(h, w_ref[...], preferred_element_type=jnp.float32) + b_ref[...]
    m = jnp.max(z, axis=1, keepdims=True)
    zz = z - m
    out_ref[...] = zz - jnp.log(jnp.sum(jnp.exp(zz), axis=1, keepdims=True))


def _output_head(acc, w2, b2, out_size):
    return pl.pallas_call(
        _k5_body,
        grid=(_NP // _BLK,),
        in_specs=[
            pl.BlockSpec((2, _BLK, _AW), lambda i: (0, i, 0)),
            pl.BlockSpec((_D, out_size), lambda i: (0, 0)),
            pl.BlockSpec((1, out_size), lambda i: (0, 0)),
        ],
        out_specs=pl.BlockSpec((_BLK, out_size), lambda i: (i, 0)),
        out_shape=jax.ShapeDtypeStruct((_NP, out_size), jnp.float32),
    )(acc, w2, b2)


_HI_MASK = jnp.int32(-65536)


def _sc_body(aug_hbm, augp_hbm, src_hbm, dst_hbm, beta_hbm, z_hbm,
             acc_out,
             acc_sh, sblk0, sblk1, dblk0, dblk1,
             srow0, srow1, srow2, srow3, drow0, drow1, drow2, drow3,
             redbuf, wbuf, beta_v,
             gs, gd, gis, gid, sc0, sc1, sc2, sc3):
    c = lax.axis_index("c")
    s = lax.axis_index("s")
    wid = c * 16 + s
    sblk = (sblk0, sblk1)
    dblk = (dblk0, dblk1)
    srow = (srow0, srow1, srow2, srow3)
    drow = (drow0, drow1, drow2, drow3)
    sc = (sc0, sc1, sc2, sc3)

    pltpu.sync_copy(beta_hbm, beta_v)

    pltpu.sync_copy(z_hbm, srow0)
    base = s * _ROWS_PER_TILE
    for k in range(_ROWS_PER_TILE // _WCHUNK):
        pltpu.sync_copy(srow0, acc_sh.at[pl.ds(base + k * _WCHUNK, _WCHUNK)])
    plsc.subcore_barrier()

    lanes16 = lax.iota(jnp.int32, 16) * 16

    def compute(sr, dr):
        @pl.loop(0, _C // 16)
        def _grp(j):
            e0 = j * 16
            for e in range(16):
                acc = None
                for g in range(4):
                    p = plsc.bitcast(dr[e0 + e, pl.ds(g * 16, 16)],
                                     jnp.int32)
                    lo = plsc.bitcast(p << 16, jnp.float32)
                    hi = plsc.bitcast(p & _HI_MASK, jnp.float32)
                    t = (sr[e0 + e, pl.ds(g * 32, 16)] * lo +
                         sr[e0 + e, pl.ds(g * 32 + 16, 16)] * hi)
                    acc = t if acc is None else acc + t
                acc = (acc * sr[e0 + e, pl.ds(_D, 16)] *
                       dr[e0 + e, pl.ds(64, 16)])
                redbuf[pl.ds(e * 16, 16)] = acc
            cos = plsc.load_gather(redbuf, [lanes16])
            for k in range(1, 16):
                cos += plsc.load_gather(redbuf, [lanes16 + k])
            ex = jnp.exp(beta_v[...] * cos)
            wbuf[...] = ex
            for e in range(16):
                exv = plsc.load_gather(wbuf, [jnp.full((16,), e, jnp.int32)])
                for k in range(_D // 16):
                    sr[e0 + e, pl.ds(k * 16, 16)] = (
                        sr[e0 + e, pl.ds(k * 16, 16)] * exv)
                sr[e0 + e, pl.ds(_D, 16)] = exv

    def fire_gather(idx_s_ref, idx_d_ref, slot):
        pltpu.async_copy(aug_hbm.at[idx_s_ref], srow[slot], gs)
        pltpu.async_copy(augp_hbm.at[idx_d_ref], drow[slot], gd)

    def wait_gather(slot):
        pltpu.make_async_copy(
            aug_hbm.at[sblk0.at[0]], srow[slot], gs).wait()
        pltpu.make_async_copy(
            augp_hbm.at[dblk0.at[0]], drow[slot], gd).wait()

    def wait_scatter(slot):
        pltpu.make_async_copy(
            srow[slot], acc_sh.at[dblk0.at[0]], sc[slot]).wait()

    pltpu.sync_copy(src_hbm.at[wid, 0], sblk0)
    pltpu.sync_copy(dst_hbm.at[wid, 0], dblk0)
    fire_gather(sblk0.at[0], dblk0.at[0], 0)
    fire_gather(sblk0.at[1], dblk0.at[1], 1)

    @pl.loop(0, _NIB // 2)
    def _outer(outer):
        for pb in range(2):
            cur_s, cur_d = sblk[pb], dblk[pb]
            nxt_s, nxt_d = sblk[1 - pb], dblk[1 - pb]

            @pl.loop(0, _IB // 4)
            def _quad(inner):
                for b in range(4):
                    sb = inner * 4 + b
                    wait_gather(b)
                    if b == 0:
                        fetch_ok = (inner == 1) if pb == 0 else \
                            (inner == 1) & (outer < _NIB // 2 - 1)

                        @pl.when(fetch_ok)
                        def _():
                            nib = outer * 2 + pb + 1
                            pltpu.async_copy(
                                src_hbm.at[wid, nib], nxt_s, gis)
                            pltpu.async_copy(
                                dst_hbm.at[wid, nib], nxt_d, gid)

                    if b < 2:
                        if pb == 0:
                            @pl.when(jnp.logical_not(
                                (outer == 0) & (inner == 0)))
                            def _():
                                wait_scatter((b + 2) % 4)
                                fire_gather(cur_s.at[sb + 2],
                                            cur_d.at[sb + 2], (b + 2) % 4)

                            @pl.when((outer == 0) & (inner == 0))
                            def _():
                                fire_gather(cur_s.at[sb + 2],
                                            cur_d.at[sb + 2], (b + 2) % 4)
                        else:
                            wait_scatter((b + 2) % 4)
                            fire_gather(cur_s.at[sb + 2],
                                        cur_d.at[sb + 2], (b + 2) % 4)
                    else:
                        @pl.when(inner < _IB // 4 - 1)
                        def _():
                            wait_scatter((b + 2) % 4)
                            fire_gather(cur_s.at[sb + 2],
                                        cur_d.at[sb + 2], (b + 2) % 4)

                        blk_x = (inner == _IB // 4 - 1) if pb == 0 else \
                            (inner == _IB // 4 - 1) & (outer < _NIB // 2 - 1)

                        @pl.when(blk_x)
                        def _():
                            wait_scatter((b + 2) % 4)
                            if b == 2:
                                pltpu.make_async_copy(
                                    src_hbm.at[wid, 0], nxt_s, gis).wait()
                                pltpu.make_async_copy(
                                    dst_hbm.at[wid, 0], nxt_d, gid).wait()
                            fire_gather(nxt_s.at[b - 2], nxt_d.at[b - 2],
                                        (b + 2) % 4)

                    compute(srow[b], drow[b])
                    pltpu.async_copy(
                        srow[b], acc_sh.at[cur_d.at[sb]], sc[b], add=True)

    for slot in (0, 1, 2, 3):
        wait_scatter(slot)

    plsc.subcore_barrier()

    for k in range(_ROWS_PER_TILE // _WCHUNK):
        pltpu.sync_copy(acc_sh.at[pl.ds(base + k * _WCHUNK, _WCHUNK)], srow0)
        pltpu.sync_copy(srow0,
                        acc_out.at[c, pl.ds(base + k * _WCHUNK, _WCHUNK)])


_sc_edge_pass = pl.kernel(
    _sc_body,
    out_type=jax.ShapeDtypeStruct((2, _NP, _AW), jnp.float32),
    mesh=plsc.VectorSubcoreMesh(core_axis_name="c", subcore_axis_name="s"),
    compiler_params=pltpu.CompilerParams(
        needs_layout_passes=False, use_tc_tiling_on_sc=False),
    scratch_types=[
        pltpu.VMEM_SHARED((_NP, _AW), jnp.float32),
        pltpu.VMEM((_IB, _C), jnp.int32),
        pltpu.VMEM((_IB, _C), jnp.int32),
        pltpu.VMEM((_IB, _C), jnp.int32),
        pltpu.VMEM((_IB, _C), jnp.int32),
        pltpu.VMEM((_C, _AW), jnp.float32),
        pltpu.VMEM((_C, _AW), jnp.float32),
        pltpu.VMEM((_C, _AW), jnp.float32),
        pltpu.VMEM((_C, _AW), jnp.float32),
        pltpu.VMEM((_C, _PW), jnp.float32),
        pltpu.VMEM((_C, _PW), jnp.float32),
        pltpu.VMEM((_C, _PW), jnp.float32),
        pltpu.VMEM((_C, _PW), jnp.float32),
        pltpu.VMEM((256,), jnp.float32),
        pltpu.VMEM((16,), jnp.float32),
        pltpu.VMEM((16,), jnp.float32),
        pltpu.SemaphoreType.DMA,
        pltpu.SemaphoreType.DMA,
        pltpu.SemaphoreType.DMA,
        pltpu.SemaphoreType.DMA,
        pltpu.SemaphoreType.DMA,
        pltpu.SemaphoreType.DMA,
        pltpu.SemaphoreType.DMA,
        pltpu.SemaphoreType.DMA,
    ],
)


@jax.jit
def kernel(features, edge_index, W1, b1, betas, W2, b2):
    out_size = W2.shape[1]
    x = jnp.pad(features, ((0, _NP - _N), (0, 0)))

    n_pad_edges = _NTILES * _EPT - _E
    pad_ids = jnp.full((n_pad_edges,), _NP - 1, dtype=jnp.int32)
    src = jnp.concatenate([edge_index[0], pad_ids]).reshape(
        _NTILES, _NIB, _IB, _C)
    dst = jnp.concatenate([edge_index[1], pad_ids]).reshape(
        _NTILES, _NIB, _IB, _C)

    zrows = jnp.zeros((_C, _AW), jnp.float32)

    aug1, augp1 = _input_matmul(x, W1, b1.reshape(1, -1))
    acc1 = _sc_edge_pass(aug1, augp1, src, dst,
                         jnp.full((16,), betas[0], jnp.float32), zrows)
    aug2, augp2 = _combine_normalize(acc1)
    acc2 = _sc_edge_pass(aug2, augp2, src, dst,
                         jnp.full((16,), betas[1], jnp.float32), zrows)
    return _output_head(acc2, W2, b2.reshape(1, -1), out_size)[:_N]
